# baseline (speedup 1.0000x reference)
import functools

import jax
import jax.numpy as jnp
from jax import lax
from jax.experimental import pallas as pl
from jax.experimental.pallas import tpu as pltpu
from jax.experimental.pallas import tpu_sc as plsc


@functools.lru_cache(maxsize=None)
def _make_gather(B, H, D, R):
    info = plsc.get_sparse_core_info()
    NC = info.num_cores
    NW = NC * info.num_subcores
    assert B % (NW * R) == 0
    rows_per_w = B // NW
    n_chunks = rows_per_w // R
    assert n_chunks >= 2 and n_chunks % 2 == 0
    mesh = plsc.VectorSubcoreMesh(core_axis_name="c", subcore_axis_name="s")

    @functools.partial(
        pl.kernel,
        mesh=mesh,
        out_type=jax.ShapeDtypeStruct((H, B, D), jnp.float32),
        scratch_types=[
            pltpu.VMEM((2, R, H), jnp.int32),
            pltpu.VMEM((2, H * R), jnp.int32),
            pltpu.VMEM((2, H * R, D), jnp.float32),
            pltpu.SemaphoreType.DMA,
            pltpu.SemaphoreType.DMA,
            pltpu.SemaphoreType.DMA,
            pltpu.SemaphoreType.DMA,
        ],
        compiler_params=pltpu.CompilerParams(use_tc_tiling_on_sc=False, needs_layout_passes=False, disable_bounds_checks=True),
    )
    def gather_kernel(idx_hbm, table_hbm, out_hbm, idx_v, off_v, rows_v,
                      gsem0, gsem1, ssem0, ssem1):
        gsems = (gsem0, gsem1)
        ssems = (ssem0, ssem1)
        wid = lax.axis_index("s") * NC + lax.axis_index("c")
        base = wid * rows_per_w
        lane = lax.iota(jnp.int32, 16)

        def stage_idx(g, b):
            pltpu.sync_copy(idx_hbm.at[pl.ds(base + g * R, R)], idx_v.at[b])
            for h0 in range(0, H, 10):
                vs = [plsc.load_gather(idx_v.at[b],
                                       [lane, jnp.full((16,), h0 + i, jnp.int32)])
                      for i in range(10)]
                for i, v in enumerate(vs):
                    off_v[b, pl.ds((h0 + i) * R, R)] = v

        def gather(b):
            return pltpu.make_async_copy(
                table_hbm.at[off_v.at[b]], rows_v.at[b], gsems[b])

        def store_h(g, b, h):
            return pltpu.make_async_copy(
                rows_v.at[b, pl.ds(h * R, R)],
                out_hbm.at[h, pl.ds(base + g * R, R)], ssems[b])

        def start_stores(g, b):
            for h in range(H):
                store_h(g, b, h).start()

        def wait_stores(g, b):
            for h in range(H):
                store_h(g, b, h).wait()

        for b in range(2):
            stage_idx(b, b)
            gather(b).start()

        @pl.loop(0, n_chunks - 2, step=2)
        def _chunks(g0):
            for b in range(2):
                g = g0 + b
                gather(b).wait()
                start_stores(g, b)
                wait_stores(g, b)
                stage_idx(g + 2, b)
                gather(b).start()

        for b in range(2):
            g = n_chunks - 2 + b
            gather(b).wait()
            start_stores(g, b)
        for b in range(2):
            wait_stores(n_chunks - 2 + b, b)

    return gather_kernel


@functools.lru_cache(maxsize=None)
def _make_format(B, H, D):
    info = plsc.get_sparse_core_info()
    NC = info.num_cores
    NW = NC * info.num_subcores
    n_tc = B // 128
    n_win = H * n_tc
    assert n_win % NW == 0
    win_per_w = n_win // NW
    assert win_per_w % 2 == 0
    mesh = plsc.VectorSubcoreMesh(core_axis_name="c", subcore_axis_name="s")

    @functools.partial(
        pl.kernel,
        mesh=mesh,
        out_type=jax.ShapeDtypeStruct((H, D, B), jnp.float32),
        scratch_types=[
            pltpu.VMEM((2, D, 128), jnp.float32),
            pltpu.VMEM((2, D, 128), jnp.float32),
            pltpu.SemaphoreType.DMA,
            pltpu.SemaphoreType.DMA,
            pltpu.SemaphoreType.DMA,
            pltpu.SemaphoreType.DMA,
        ],
        compiler_params=pltpu.CompilerParams(use_tc_tiling_on_sc=True, needs_layout_passes=False, disable_bounds_checks=True),
    )
    def format_kernel(x_hbm, out_hbm, g_v, o_v, gsem0, gsem1, ssem0, ssem1):
        gsems = (gsem0, gsem1)
        ssems = (ssem0, ssem1)
        wid = lax.axis_index("s") * NC + lax.axis_index("c")
        w0 = wid * win_per_w
        lane = lax.iota(jnp.int32, 16)
        row_pat = lax.shift_right_logical(lane, 2)
        col_pat = lax.bitwise_and(lane, jnp.full((16,), 3, jnp.int32)) * 32

        def load(w, b):
            h = w // n_tc
            tc = w % n_tc
            row0 = h * (B * D // 128) + tc * (128 * D // 128)
            return pltpu.make_async_copy(
                x_hbm.at[pl.ds(row0, D)], g_v.at[b], gsems[b])

        def store(w, b):
            h = w // n_tc
            tc = w % n_tc
            return pltpu.make_async_copy(
                o_v.at[b], out_hbm.at[h, :, pl.ds(tc * 128, 128)], ssems[b])

        lane_hi = lane + 16

        def shuffle(b):
            for l in range(128):
                r = l // 4
                m = (l % 4) * 32
                v1 = g_v[b, r, pl.ds(m, 16)]
                v2 = g_v[b, r, pl.ds(m + 16, 16)]
                li = jnp.full((16,), l, jnp.int32)
                plsc.store_scatter(o_v.at[b], [lane, li], v1)
                plsc.store_scatter(o_v.at[b], [lane_hi, li], v2)

        for b in range(2):
            load(w0 + b, b).start()

        @pl.loop(0, win_per_w - 2, step=2)
        def _wins(i0):
            for b in range(2):
                w = w0 + i0 + b
                load(w, b).wait()

                @pl.when(i0 + b >= 2)
                def _():
                    store(w - 2, b).wait()

                shuffle(b)
                load(w + 2, b).start()
                store(w, b).start()

        for b in range(2):
            w = w0 + win_per_w - 2 + b

            @pl.when(win_per_w > 2)
            def _():
                store(w - 2, b).wait()

            load(w, b).wait()
            shuffle(b)
            store(w, b).start()
        for b in range(2):
            store(w0 + win_per_w - 2 + b, b).wait()

    return format_kernel


def kernel(indices, table):
    B, H = indices.shape
    V, D = table.shape
    g2 = _make_gather(B, H, D, 16)(indices.astype(jnp.int32), table)
    x = g2.reshape(H * B * D // 128, 128)
    o = _make_format(B, H, D)(x)
    return jnp.transpose(o, (2, 0, 1))


# speedup vs baseline: 1.0615x; 1.0615x over previous
import functools

import jax
import jax.numpy as jnp
from jax import lax
from jax.experimental import pallas as pl
from jax.experimental.pallas import tpu as pltpu
from jax.experimental.pallas import tpu_sc as plsc


@functools.lru_cache(maxsize=None)
def _make_gather(B, H, D, R):
    info = plsc.get_sparse_core_info()
    NC = info.num_cores
    NW = NC * info.num_subcores
    assert B % (NW * R) == 0
    rows_per_w = B // NW
    n_chunks = rows_per_w // R
    assert n_chunks >= 2 and n_chunks % 2 == 0
    mesh = plsc.VectorSubcoreMesh(core_axis_name="c", subcore_axis_name="s")

    @functools.partial(
        pl.kernel,
        mesh=mesh,
        out_type=jax.ShapeDtypeStruct((B, H, D), jnp.float32),
        scratch_types=[
            pltpu.VMEM((2, R, H), jnp.int32),
            pltpu.VMEM((2, R, H, D), jnp.float32),
            pltpu.SemaphoreType.DMA,
            pltpu.SemaphoreType.DMA,
            pltpu.SemaphoreType.DMA,
            pltpu.SemaphoreType.DMA,
        ],
        compiler_params=pltpu.CompilerParams(use_tc_tiling_on_sc=False),
    )
    def gather_kernel(idx_hbm, table_hbm, out_hbm, idx_v, rows_v,
                      gsem0, gsem1, ssem0, ssem1):
        gsems = (gsem0, gsem1)
        ssems = (ssem0, ssem1)
        wid = lax.axis_index("s") * NC + lax.axis_index("c")
        base = wid * rows_per_w

        def stage_idx(g, b):
            pltpu.sync_copy(idx_hbm.at[pl.ds(base + g * R, R)], idx_v.at[b])

        def row_gather(b, i):
            return pltpu.make_async_copy(
                table_hbm.at[idx_v.at[b, i]], rows_v.at[b, i], gsems[b])

        def start_gathers(b):
            for i in range(R):
                row_gather(b, i).start()

        def wait_gathers(b):
            for i in range(R):
                row_gather(b, i).wait()

        def store(g, b):
            return pltpu.make_async_copy(
                rows_v.at[b], out_hbm.at[pl.ds(base + g * R, R)], ssems[b])

        for b in range(2):
            stage_idx(b, b)
            start_gathers(b)

        @pl.loop(0, n_chunks - 2, step=2)
        def _chunks(g0):
            for b in range(2):
                g = g0 + b
                wait_gathers(b)
                store(g, b).start()
                store(g, b).wait()
                stage_idx(g + 2, b)
                start_gathers(b)

        for b in range(2):
            g = n_chunks - 2 + b
            wait_gathers(b)
            store(g, b).start()
        for b in range(2):
            store(n_chunks - 2 + b, b).wait()

    return gather_kernel


def kernel(indices, table):
    B, H = indices.shape
    V, D = table.shape
    table_r = jnp.pad(table, ((0, 0), (0, 128 - D))).reshape(4 * V, D)
    idx4 = indices.astype(jnp.int32) * 4
    return _make_gather(B, H, D, 16)(idx4, table_r)
